# baseline (speedup 1.0000x reference)
import functools
import math

import jax
import jax.numpy as jnp
from jax import lax
from jax.experimental import pallas as pl
from jax.experimental.pallas import tpu as pltpu
from jax.experimental.pallas import tpu_sc as plsc

_N = 10000
_E = 160000
_D = 128
_DA = 16
_EPS = 1e-9

_BN = 200
_BE = 640
_GCH = 200
_SCH = 200


def _rownorm(x):
    m = jnp.mean(x, axis=1, keepdims=True)
    var = jnp.sum((x - m) ** 2, axis=1, keepdims=True) / (_D - 1)
    return x / (jnp.sqrt(var) + _EPS)


def _silu(x):
    return x * jax.nn.sigmoid(x)


def _node_mix_body(xn_ref, attr_ref, wbT_ref, wA_ref, wB_ref, wC_ref, bl_ref,
                   out_ref):
    x1 = xn_ref[...]
    x2 = attr_ref[...]
    p = jnp.concatenate([x2[:, j:j + 1] * x1 for j in range(_DA)], axis=1)
    xbi = jnp.dot(p, wbT_ref[...], preferred_element_type=jnp.float32)
    x = (jnp.dot(x1, wA_ref[...], preferred_element_type=jnp.float32)
         + jnp.dot(x2, wB_ref[...], preferred_element_type=jnp.float32)
         + jnp.dot(xbi, wC_ref[...], preferred_element_type=jnp.float32)
         + bl_ref[...])
    out_ref[...] = _rownorm(x)


def _colnorm(x):
    m = jnp.mean(x, axis=0, keepdims=True)
    var = jnp.sum((x - m) ** 2, axis=0, keepdims=True) / (_D - 1)
    return x / (jnp.sqrt(var) + _EPS)


def _bilinear_bf16_t(uT, vT, w2_ref):
    n = uT.shape[1]
    xbiT = jnp.zeros((_D, n), dtype=jnp.float32)
    for g in range(16):
        blocks = [uT[i:i + 1, :] * vT for i in range(8 * g, 8 * g + 8)]
        pT = jnp.concatenate(blocks, axis=0).astype(jnp.bfloat16)
        xbiT += jnp.dot(w2_ref[:, 8 * g * _D:(8 * g + 8) * _D], pT,
                        preferred_element_type=jnp.float32)
    return xbiT


def _edge_body(s_ref, d_ref, ea_ref, wf1_ref, bf1_ref, wb2_ref,
               wnA_ref, wnB_ref, wnC_ref, bln_ref,
               wbxe_ref, wxA_ref, wxb_ref, wxC_ref, blx_ref,
               wf2_ref, bf2_ref, out_ref):
    sT = s_ref[...].T
    dT = d_ref[...].T
    ea = ea_ref[...]
    w = _silu(ea * wf1_ref[...] + bf1_ref[...])
    uT = w * (sT - dT)
    vT = w * (sT + dT) * 0.5
    xbiT = _bilinear_bf16_t(uT, vT, wb2_ref)
    xeT = (jnp.dot(wnA_ref[...], uT, preferred_element_type=jnp.float32)
           + jnp.dot(wnB_ref[...], vT, preferred_element_type=jnp.float32)
           + jnp.dot(wnC_ref[...], xbiT, preferred_element_type=jnp.float32)
           + bln_ref[...])
    xbi2T = jnp.dot(wbxe_ref[...], xeT,
                    preferred_element_type=jnp.float32) * ea
    xe2T = (jnp.dot(wxA_ref[...], xeT, preferred_element_type=jnp.float32)
            + wxb_ref[...] * ea
            + jnp.dot(wxC_ref[...], xbi2T, preferred_element_type=jnp.float32)
            + blx_ref[...])
    xe2T = _colnorm(xe2T)
    w2 = _silu(ea * wf2_ref[...] + bf2_ref[...])
    out_ref[...] = (w2 * xe2T).T


def _final_body(x1_ref, x2_ref, wb2_ref, wA_ref, wB_ref, wC_ref,
                bl_ref, out_ref):
    nrm = 1.0 / math.sqrt(20.0)
    x1T = x1_ref[...].T
    x2T = x2_ref[...].T
    aT = (x1T - x2T) * nrm
    bT = (x1T + x2T) * nrm
    xbiT = _bilinear_bf16_t(aT, bT, wb2_ref)
    xT = (jnp.dot(wA_ref[...], aT, preferred_element_type=jnp.float32)
          + jnp.dot(wB_ref[...], bT, preferred_element_type=jnp.float32)
          + jnp.dot(wC_ref[...], xbiT, preferred_element_type=jnp.float32)
          + bl_ref[...])
    out_ref[...] = _colnorm(_silu(xT)).T


def _full(shape):
    return pl.BlockSpec(shape, lambda i: tuple(0 for _ in shape))


def _make_gather():
    mesh = plsc.VectorSubcoreMesh(core_axis_name="c", subcore_axis_name="s")
    nw = 32
    per_w = _E // nw
    nch = per_w // _GCH

    @functools.partial(
        pl.kernel, mesh=mesh,
        out_type=(jax.ShapeDtypeStruct((_E, _D), jnp.float32),
                  jax.ShapeDtypeStruct((_E, _D), jnp.float32)),
        scratch_types=[pltpu.VMEM((_GCH,), jnp.int32),
                       pltpu.VMEM((_GCH, _D), jnp.float32),
                       pltpu.VMEM((_GCH,), jnp.int32),
                       pltpu.VMEM((_GCH, _D), jnp.float32),
                       pltpu.SemaphoreType.DMA,
                       pltpu.SemaphoreType.DMA],
    )
    def gather(xn_hbm, src_hbm, dst_hbm, outs_hbm, outd_hbm,
               idx1_v, rows1_v, idx2_v, rows2_v, sem1, sem2):
        wid = lax.axis_index("s") * 2 + lax.axis_index("c")
        base = wid * per_w

        def body(k, carry):
            off = base + k * _GCH
            pltpu.sync_copy(src_hbm.at[pl.ds(off, _GCH)], idx1_v)
            cp1 = pltpu.async_copy(xn_hbm.at[idx1_v], rows1_v, sem1)
            pltpu.sync_copy(dst_hbm.at[pl.ds(off, _GCH)], idx2_v)
            cp2 = pltpu.async_copy(xn_hbm.at[idx2_v], rows2_v, sem2)
            cp1.wait()
            pltpu.sync_copy(rows1_v, outs_hbm.at[pl.ds(off, _GCH)])
            cp2.wait()
            pltpu.sync_copy(rows2_v, outd_hbm.at[pl.ds(off, _GCH)])
            return carry

        lax.fori_loop(0, nch, body, 0)

    return gather


def _make_scatter():
    mesh = plsc.VectorSubcoreMesh(core_axis_name="c", subcore_axis_name="s")
    per_t = _E // 16
    nch = per_t // _SCH
    nzc = _N // _SCH

    @functools.partial(
        pl.kernel, mesh=mesh,
        out_type=(jax.ShapeDtypeStruct((_N, _D), jnp.float32),
                  jax.ShapeDtypeStruct((_N, _D), jnp.float32)),
        scratch_types=[pltpu.VMEM((_SCH,), jnp.int32),
                       pltpu.VMEM((_SCH, _D), jnp.float32),
                       pltpu.VMEM_SHARED((_N, _D), jnp.float32)],
    )
    def scatter(y_hbm, dst_hbm, src_hbm, zb_hbm, out1_hbm, out2_hbm,
                idx_v, y_v, acc_sh):
        cid = lax.axis_index("c")
        sid = lax.axis_index("s")

        for t in range((nzc + 15) // 16):
            kk = sid + t * 16

            @pl.when(kk < nzc)
            def _():
                pltpu.sync_copy(zb_hbm, acc_sh.at[pl.ds(kk * _SCH, _SCH)])

        plsc.subcore_barrier()

        def chunk(idx_hbm, k):
            off = sid * per_t + k * _SCH
            pltpu.sync_copy(idx_hbm.at[pl.ds(off, _SCH)], idx_v)
            pltpu.sync_copy(y_hbm.at[pl.ds(off, _SCH)], y_v)
            pltpu.sync_copy(y_v, acc_sh.at[idx_v], add=True)

        @pl.when(cid == 0)
        def _():
            lax.fori_loop(0, nch, lambda k, c: (chunk(dst_hbm, k), c)[1], 0)

        @pl.when(cid == 1)
        def _():
            lax.fori_loop(0, nch, lambda k, c: (chunk(src_hbm, k), c)[1], 0)

        plsc.subcore_barrier()

        for t in range((nzc + 15) // 16):
            kk = sid + t * 16

            @pl.when(kk < nzc)
            def _():
                sl = pl.ds(kk * _SCH, _SCH)

                @pl.when(cid == 0)
                def _():
                    pltpu.sync_copy(acc_sh.at[sl], out1_hbm.at[sl])

                @pl.when(cid == 1)
                def _():
                    pltpu.sync_copy(acc_sh.at[sl], out2_hbm.at[sl])

    return scatter


@functools.lru_cache(maxsize=1)
def _sc_kernels():
    return _make_gather(), _make_scatter()


def kernel(xn, xn_attr, xe_attr, xe_src, xe_dst, Wb_xn, Wl_xn, bl_xn,
           W_fc1, b_fc1, Wb_n2e, Wl_n2e, bl_n2e, Wb_xe, Wl_xe, bl_xe,
           W_fc2, b_fc2, Wb_e2n, Wl_e2n, bl_e2n):
    f32 = jnp.float32
    wbxnT = Wb_xn.transpose(2, 1, 0).reshape(_DA * _D, _D)
    wxnA = Wl_xn[:, :_D].T
    wxnB = Wl_xn[:, _D:_D + _DA].T
    wxnC = Wl_xn[:, _D + _DA:].T
    blxn = bl_xn.reshape(1, _D)
    wbn2e2 = Wb_n2e.reshape(_D, _D * _D).astype(jnp.bfloat16)
    wnA = Wl_n2e[:, :_D]
    wnB = Wl_n2e[:, _D:2 * _D]
    wnC = Wl_n2e[:, 2 * _D:]
    bln = bl_n2e.reshape(_D, 1)
    wbxe = Wb_xe[:, :, 0]
    wxA = Wl_xe[:, :_D]
    wxb = Wl_xe[:, _D].reshape(_D, 1)
    wxC = Wl_xe[:, _D + 1:]
    blx = bl_xe.reshape(_D, 1)
    wbe2n2 = Wb_e2n.reshape(_D, _D * _D).astype(jnp.bfloat16)
    weA = Wl_e2n[:, :_D]
    weB = Wl_e2n[:, _D:2 * _D]
    weC = Wl_e2n[:, 2 * _D:]
    ble = bl_e2n.reshape(_D, 1)
    wf1 = W_fc1.reshape(_D, 1)
    bf1 = b_fc1.reshape(_D, 1)
    wf2 = W_fc2.reshape(_D, 1)
    bf2 = b_fc2.reshape(_D, 1)
    ea_row = xe_attr.reshape(1, _E)
    src = xe_src.astype(jnp.int32)
    dst = xe_dst.astype(jnp.int32)

    xn_m = pl.pallas_call(
        _node_mix_body,
        grid=(_N // _BN,),
        in_specs=[pl.BlockSpec((_BN, _D), lambda i: (i, 0)),
                  pl.BlockSpec((_BN, _DA), lambda i: (i, 0)),
                  _full((_DA * _D, _D)), _full((_D, _D)), _full((_DA, _D)),
                  _full((_D, _D)), _full((1, _D))],
        out_specs=pl.BlockSpec((_BN, _D), lambda i: (i, 0)),
        out_shape=jax.ShapeDtypeStruct((_N, _D), f32),
    )(xn, xn_attr, wbxnT, wxnA, wxnB, wxnC, blxn)

    _gather, _scatter = _sc_kernels()
    s_rows, d_rows = _gather(xn_m, src, dst)

    y = pl.pallas_call(
        _edge_body,
        grid=(_E // _BE,),
        in_specs=[pl.BlockSpec((_BE, _D), lambda i: (i, 0)),
                  pl.BlockSpec((_BE, _D), lambda i: (i, 0)),
                  pl.BlockSpec((1, _BE), lambda i: (0, i)),
                  _full((_D, 1)), _full((_D, 1)),
                  _full((_D, _D * _D)),
                  _full((_D, _D)), _full((_D, _D)), _full((_D, _D)),
                  _full((_D, 1)),
                  _full((_D, _D)), _full((_D, _D)), _full((_D, 1)),
                  _full((_D, _D)), _full((_D, 1)),
                  _full((_D, 1)), _full((_D, 1))],
        out_specs=pl.BlockSpec((_BE, _D), lambda i: (i, 0)),
        out_shape=jax.ShapeDtypeStruct((_E, _D), f32),
    )(s_rows, d_rows, ea_row, wf1, bf1, wbn2e2, wnA, wnB, wnC, bln,
      wbxe, wxA, wxb, wxC, blx, wf2, bf2)

    zb = jnp.zeros((_SCH, _D), f32)
    xn1, xn2 = _scatter(y, dst, src, zb)

    out = pl.pallas_call(
        _final_body,
        grid=(_N // _BN,),
        in_specs=[pl.BlockSpec((_BN, _D), lambda i: (i, 0)),
                  pl.BlockSpec((_BN, _D), lambda i: (i, 0)),
                  _full((_D, _D * _D)),
                  _full((_D, _D)), _full((_D, _D)), _full((_D, _D)),
                  _full((_D, 1))],
        out_specs=pl.BlockSpec((_BN, _D), lambda i: (i, 0)),
        out_shape=jax.ShapeDtypeStruct((_N, _D), f32),
    )(xn1, xn2, wbe2n2, weA, weB, weC, ble)

    return out


# speedup vs baseline: 1.0870x; 1.0870x over previous
import functools
import math

import jax
import jax.numpy as jnp
from jax import lax
from jax.experimental import pallas as pl
from jax.experimental.pallas import tpu as pltpu
from jax.experimental.pallas import tpu_sc as plsc

_N = 10000
_E = 160000
_D = 128
_DA = 16
_EPS = 1e-9

_BN = 200
_BE = 640
_GCH = 200
_SCH = 200


def _rownorm(x):
    m = jnp.mean(x, axis=1, keepdims=True)
    var = jnp.sum((x - m) ** 2, axis=1, keepdims=True) / (_D - 1)
    return x / (jnp.sqrt(var) + _EPS)


def _silu(x):
    return x * jax.nn.sigmoid(x)


def _node_mix_body(xn_ref, attr_ref, wbT_ref, wA_ref, wB_ref, wC_ref, bl_ref,
                   out_ref):
    x1 = xn_ref[...]
    x2 = attr_ref[...]
    p = jnp.concatenate([x2[:, j:j + 1] * x1 for j in range(_DA)], axis=1)
    xbi = jnp.dot(p, wbT_ref[...], preferred_element_type=jnp.float32)
    x = (jnp.dot(x1, wA_ref[...], preferred_element_type=jnp.float32)
         + jnp.dot(x2, wB_ref[...], preferred_element_type=jnp.float32)
         + jnp.dot(xbi, wC_ref[...], preferred_element_type=jnp.float32)
         + bl_ref[...])
    out_ref[...] = _rownorm(x)


def _colnorm(x):
    m = jnp.mean(x, axis=0, keepdims=True)
    var = jnp.sum((x - m) ** 2, axis=0, keepdims=True) / (_D - 1)
    return x / (jnp.sqrt(var) + _EPS)


def _bilinear_bf16_t(uT, vT, w2_ref):
    n = uT.shape[1]
    xbiT = jnp.zeros((_D, n), dtype=jnp.float32)
    for g in range(16):
        blocks = [uT[i:i + 1, :] * vT for i in range(8 * g, 8 * g + 8)]
        pT = jnp.concatenate(blocks, axis=0).astype(jnp.bfloat16)
        xbiT += jnp.dot(w2_ref[:, 8 * g * _D:(8 * g + 8) * _D], pT,
                        preferred_element_type=jnp.float32)
    return xbiT


def _edge_body(s_ref, d_ref, ea_ref, wf1_ref, bf1_ref, wb2_ref,
               wnA_ref, wnB_ref, wnC_ref, bln_ref,
               wbxe_ref, wxA_ref, wxb_ref, wxC_ref, blx_ref,
               wf2_ref, bf2_ref, out_ref):
    sT = s_ref[...].T
    dT = d_ref[...].T
    ea = ea_ref[...]
    w = _silu(ea * wf1_ref[...] + bf1_ref[...])
    uT = w * (sT - dT)
    vT = w * (sT + dT) * 0.5
    xbiT = _bilinear_bf16_t(uT, vT, wb2_ref)
    xeT = (jnp.dot(wnA_ref[...], uT, preferred_element_type=jnp.float32)
           + jnp.dot(wnB_ref[...], vT, preferred_element_type=jnp.float32)
           + jnp.dot(wnC_ref[...], xbiT, preferred_element_type=jnp.float32)
           + bln_ref[...])
    xbi2T = jnp.dot(wbxe_ref[...], xeT,
                    preferred_element_type=jnp.float32) * ea
    xe2T = (jnp.dot(wxA_ref[...], xeT, preferred_element_type=jnp.float32)
            + wxb_ref[...] * ea
            + jnp.dot(wxC_ref[...], xbi2T, preferred_element_type=jnp.float32)
            + blx_ref[...])
    xe2T = _colnorm(xe2T)
    w2 = _silu(ea * wf2_ref[...] + bf2_ref[...])
    out_ref[...] = (w2 * xe2T).T


def _final_body(x1a_ref, x1b_ref, x2a_ref, x2b_ref, wb2_ref,
                wA_ref, wB_ref, wC_ref, bl_ref, out_ref):
    nrm = 1.0 / math.sqrt(20.0)
    x1T = (x1a_ref[...] + x1b_ref[...]).T
    x2T = (x2a_ref[...] + x2b_ref[...]).T
    aT = (x1T - x2T) * nrm
    bT = (x1T + x2T) * nrm
    xbiT = _bilinear_bf16_t(aT, bT, wb2_ref)
    xT = (jnp.dot(wA_ref[...], aT, preferred_element_type=jnp.float32)
          + jnp.dot(wB_ref[...], bT, preferred_element_type=jnp.float32)
          + jnp.dot(wC_ref[...], xbiT, preferred_element_type=jnp.float32)
          + bl_ref[...])
    out_ref[...] = _colnorm(_silu(xT)).T


def _full(shape):
    return pl.BlockSpec(shape, lambda i: tuple(0 for _ in shape))


def _make_gather(e_tot, chunk):
    mesh = plsc.VectorSubcoreMesh(core_axis_name="c", subcore_axis_name="s")
    nw = 32
    per_w = e_tot // nw
    nch = per_w // chunk

    @functools.partial(
        pl.kernel, mesh=mesh,
        out_type=(jax.ShapeDtypeStruct((e_tot, _D), jnp.float32),
                  jax.ShapeDtypeStruct((e_tot, _D), jnp.float32)),
        scratch_types=[pltpu.VMEM((chunk,), jnp.int32),
                       pltpu.VMEM((chunk, _D), jnp.float32),
                       pltpu.VMEM((chunk,), jnp.int32),
                       pltpu.VMEM((chunk, _D), jnp.float32),
                       pltpu.SemaphoreType.DMA,
                       pltpu.SemaphoreType.DMA],
    )
    def gather(xn_hbm, src_hbm, dst_hbm, outs_hbm, outd_hbm,
               idx1_v, rows1_v, idx2_v, rows2_v, sem1, sem2):
        wid = lax.axis_index("s") * 2 + lax.axis_index("c")
        base = wid * per_w

        def body(k, carry):
            off = base + k * chunk
            pltpu.sync_copy(src_hbm.at[pl.ds(off, chunk)], idx1_v)
            cp1 = pltpu.async_copy(xn_hbm.at[idx1_v], rows1_v, sem1)
            pltpu.sync_copy(dst_hbm.at[pl.ds(off, chunk)], idx2_v)
            cp2 = pltpu.async_copy(xn_hbm.at[idx2_v], rows2_v, sem2)
            cp1.wait()
            pltpu.sync_copy(rows1_v, outs_hbm.at[pl.ds(off, chunk)])
            cp2.wait()
            pltpu.sync_copy(rows2_v, outd_hbm.at[pl.ds(off, chunk)])
            return carry

        lax.fori_loop(0, nch, body, 0)

    return gather


def _make_scatter(e_tot, chunk):
    mesh = plsc.VectorSubcoreMesh(core_axis_name="c", subcore_axis_name="s")
    per_t = e_tot // 16
    nch = per_t // chunk
    nzc = _N // _SCH

    @functools.partial(
        pl.kernel, mesh=mesh,
        out_type=(jax.ShapeDtypeStruct((_N, _D), jnp.float32),
                  jax.ShapeDtypeStruct((_N, _D), jnp.float32)),
        scratch_types=[pltpu.VMEM((chunk,), jnp.int32),
                       pltpu.VMEM((chunk, _D), jnp.float32),
                       pltpu.VMEM_SHARED((_N, _D), jnp.float32)],
    )
    def scatter(y_hbm, dst_hbm, src_hbm, zb_hbm, out1_hbm, out2_hbm,
                idx_v, y_v, acc_sh):
        cid = lax.axis_index("c")
        sid = lax.axis_index("s")

        for t in range((nzc + 15) // 16):
            kk = sid + t * 16

            @pl.when(kk < nzc)
            def _():
                pltpu.sync_copy(zb_hbm, acc_sh.at[pl.ds(kk * _SCH, _SCH)])

        plsc.subcore_barrier()

        def do_chunk(idx_hbm, k):
            off = sid * per_t + k * chunk
            pltpu.sync_copy(idx_hbm.at[pl.ds(off, chunk)], idx_v)
            pltpu.sync_copy(y_hbm.at[pl.ds(off, chunk)], y_v)
            pltpu.sync_copy(y_v, acc_sh.at[idx_v], add=True)

        @pl.when(cid == 0)
        def _():
            lax.fori_loop(0, nch, lambda k, c: (do_chunk(dst_hbm, k), c)[1], 0)

        @pl.when(cid == 1)
        def _():
            lax.fori_loop(0, nch, lambda k, c: (do_chunk(src_hbm, k), c)[1], 0)

        plsc.subcore_barrier()

        for t in range((nzc + 15) // 16):
            kk = sid + t * 16

            @pl.when(kk < nzc)
            def _():
                sl = pl.ds(kk * _SCH, _SCH)

                @pl.when(cid == 0)
                def _():
                    pltpu.sync_copy(acc_sh.at[sl], out1_hbm.at[sl])

                @pl.when(cid == 1)
                def _():
                    pltpu.sync_copy(acc_sh.at[sl], out2_hbm.at[sl])

    return scatter


_EA = 126 * 640
_EB = _E - _EA


@functools.lru_cache(maxsize=1)
def _sc_kernels():
    return (_make_gather(_EA, 168), _make_gather(_EB, 248),
            _make_scatter(_EA, 240), _make_scatter(_EB, 248))


def kernel(xn, xn_attr, xe_attr, xe_src, xe_dst, Wb_xn, Wl_xn, bl_xn,
           W_fc1, b_fc1, Wb_n2e, Wl_n2e, bl_n2e, Wb_xe, Wl_xe, bl_xe,
           W_fc2, b_fc2, Wb_e2n, Wl_e2n, bl_e2n):
    f32 = jnp.float32
    wbxnT = Wb_xn.transpose(2, 1, 0).reshape(_DA * _D, _D)
    wxnA = Wl_xn[:, :_D].T
    wxnB = Wl_xn[:, _D:_D + _DA].T
    wxnC = Wl_xn[:, _D + _DA:].T
    blxn = bl_xn.reshape(1, _D)
    wbn2e2 = Wb_n2e.reshape(_D, _D * _D).astype(jnp.bfloat16)
    wnA = Wl_n2e[:, :_D]
    wnB = Wl_n2e[:, _D:2 * _D]
    wnC = Wl_n2e[:, 2 * _D:]
    bln = bl_n2e.reshape(_D, 1)
    wbxe = Wb_xe[:, :, 0]
    wxA = Wl_xe[:, :_D]
    wxb = Wl_xe[:, _D].reshape(_D, 1)
    wxC = Wl_xe[:, _D + 1:]
    blx = bl_xe.reshape(_D, 1)
    wbe2n2 = Wb_e2n.reshape(_D, _D * _D).astype(jnp.bfloat16)
    weA = Wl_e2n[:, :_D]
    weB = Wl_e2n[:, _D:2 * _D]
    weC = Wl_e2n[:, 2 * _D:]
    ble = bl_e2n.reshape(_D, 1)
    wf1 = W_fc1.reshape(_D, 1)
    bf1 = b_fc1.reshape(_D, 1)
    wf2 = W_fc2.reshape(_D, 1)
    bf2 = b_fc2.reshape(_D, 1)
    ea_row = xe_attr.reshape(1, _E)
    src = xe_src.astype(jnp.int32)
    dst = xe_dst.astype(jnp.int32)

    xn_m = pl.pallas_call(
        _node_mix_body,
        grid=(_N // _BN,),
        in_specs=[pl.BlockSpec((_BN, _D), lambda i: (i, 0)),
                  pl.BlockSpec((_BN, _DA), lambda i: (i, 0)),
                  _full((_DA * _D, _D)), _full((_D, _D)), _full((_DA, _D)),
                  _full((_D, _D)), _full((1, _D))],
        out_specs=pl.BlockSpec((_BN, _D), lambda i: (i, 0)),
        out_shape=jax.ShapeDtypeStruct((_N, _D), f32),
    )(xn, xn_attr, wbxnT, wxnA, wxnB, wxnC, blxn)

    _gatherA, _gatherB, _scatterA, _scatterB = _sc_kernels()
    srcA, srcB = src[:_EA], src[_EA:]
    dstA, dstB = dst[:_EA], dst[_EA:]
    sA, dA = _gatherA(xn_m, srcA, dstA)
    sB, dB = _gatherB(xn_m, srcB, dstB)

    def edge(s_rows, d_rows, ea, ne):
        return pl.pallas_call(
            _edge_body,
            grid=(ne // _BE,),
            in_specs=[pl.BlockSpec((_BE, _D), lambda i: (i, 0)),
                      pl.BlockSpec((_BE, _D), lambda i: (i, 0)),
                      pl.BlockSpec((1, _BE), lambda i: (0, i)),
                      _full((_D, 1)), _full((_D, 1)),
                      _full((_D, _D * _D)),
                      _full((_D, _D)), _full((_D, _D)), _full((_D, _D)),
                      _full((_D, 1)),
                      _full((_D, _D)), _full((_D, _D)), _full((_D, 1)),
                      _full((_D, _D)), _full((_D, 1)),
                      _full((_D, 1)), _full((_D, 1))],
            out_specs=pl.BlockSpec((_BE, _D), lambda i: (i, 0)),
            out_shape=jax.ShapeDtypeStruct((ne, _D), f32),
        )(s_rows, d_rows, ea, wf1, bf1, wbn2e2, wnA, wnB, wnC, bln,
          wbxe, wxA, wxb, wxC, blx, wf2, bf2)

    yA = edge(sA, dA, ea_row[:, :_EA], _EA)
    yB = edge(sB, dB, ea_row[:, _EA:], _EB)

    zb = jnp.zeros((_SCH, _D), f32)
    x1a, x2a = _scatterA(yA, dstA, srcA, zb)
    x1b, x2b = _scatterB(yB, dstB, srcB, zb)

    out = pl.pallas_call(
        _final_body,
        grid=(_N // _BN,),
        in_specs=[pl.BlockSpec((_BN, _D), lambda i: (i, 0)),
                  pl.BlockSpec((_BN, _D), lambda i: (i, 0)),
                  pl.BlockSpec((_BN, _D), lambda i: (i, 0)),
                  pl.BlockSpec((_BN, _D), lambda i: (i, 0)),
                  _full((_D, _D * _D)),
                  _full((_D, _D)), _full((_D, _D)), _full((_D, _D)),
                  _full((_D, 1))],
        out_specs=pl.BlockSpec((_BN, _D), lambda i: (i, 0)),
        out_shape=jax.ShapeDtypeStruct((_N, _D), f32),
    )(x1a, x1b, x2a, x2b, wbe2n2, weA, weB, weC, ble)

    return out


# speedup vs baseline: 1.2616x; 1.1606x over previous
import functools
import math

import jax
import jax.numpy as jnp
from jax import lax
from jax.experimental import pallas as pl
from jax.experimental.pallas import tpu as pltpu
from jax.experimental.pallas import tpu_sc as plsc

_N = 10000
_E = 160000
_D = 128
_DA = 16
_EPS = 1e-9

_BN = 200
_BE = 1280
_GCH = 200
_SCH = 200


def _rownorm(x):
    m = jnp.mean(x, axis=1, keepdims=True)
    var = jnp.sum((x - m) ** 2, axis=1, keepdims=True) / (_D - 1)
    return x / (jnp.sqrt(var) + _EPS)


def _silu(x):
    return x * jax.nn.sigmoid(x)


def _node_mix_body(xn_ref, attr_ref, wbT_ref, wA_ref, wB_ref, wC_ref, bl_ref,
                   out_ref):
    x1 = xn_ref[...]
    x2 = attr_ref[...]
    p = jnp.concatenate([x2[:, j:j + 1] * x1 for j in range(_DA)], axis=1)
    xbi = jnp.dot(p, wbT_ref[...], preferred_element_type=jnp.float32)
    x = (jnp.dot(x1, wA_ref[...], preferred_element_type=jnp.float32)
         + jnp.dot(x2, wB_ref[...], preferred_element_type=jnp.float32)
         + jnp.dot(xbi, wC_ref[...], preferred_element_type=jnp.float32)
         + bl_ref[...])
    out_ref[...] = _rownorm(x)


def _colnorm(x):
    m = jnp.mean(x, axis=0, keepdims=True)
    var = jnp.sum((x - m) ** 2, axis=0, keepdims=True) / (_D - 1)
    return x / (jnp.sqrt(var) + _EPS)


def _bilinear_bf16_t(uT, vT, w2_ref):
    n = uT.shape[1]
    xbiT = jnp.zeros((_D, n), dtype=jnp.float32)
    for g in range(16):
        blocks = [uT[i:i + 1, :] * vT for i in range(8 * g, 8 * g + 8)]
        pT = jnp.concatenate(blocks, axis=0).astype(jnp.bfloat16)
        xbiT += jnp.dot(w2_ref[:, 8 * g * _D:(8 * g + 8) * _D], pT,
                        preferred_element_type=jnp.float32)
    return xbiT


def _edge_body(s_ref, d_ref, ea_ref, wf1_ref, bf1_ref, wb2_ref,
               wnA_ref, wnB_ref, wnC_ref, bln_ref,
               wbxe_ref, wxA_ref, wxb_ref, wxC_ref, blx_ref,
               wf2_ref, bf2_ref, out_ref):
    sT = s_ref[...].T
    dT = d_ref[...].T
    ea = ea_ref[...]
    w = _silu(ea * wf1_ref[...] + bf1_ref[...])
    uT = w * (sT - dT)
    vT = w * (sT + dT) * 0.5
    xbiT = _bilinear_bf16_t(uT, vT, wb2_ref)
    xeT = (jnp.dot(wnA_ref[...], uT, preferred_element_type=jnp.float32)
           + jnp.dot(wnB_ref[...], vT, preferred_element_type=jnp.float32)
           + jnp.dot(wnC_ref[...], xbiT, preferred_element_type=jnp.float32)
           + bln_ref[...])
    xbi2T = jnp.dot(wbxe_ref[...], xeT,
                    preferred_element_type=jnp.float32) * ea
    xe2T = (jnp.dot(wxA_ref[...], xeT, preferred_element_type=jnp.float32)
            + wxb_ref[...] * ea
            + jnp.dot(wxC_ref[...], xbi2T, preferred_element_type=jnp.float32)
            + blx_ref[...])
    xe2T = _colnorm(xe2T)
    w2 = _silu(ea * wf2_ref[...] + bf2_ref[...])
    out_ref[...] = (w2 * xe2T).T


def _final_body(x1a_ref, x1b_ref, x2a_ref, x2b_ref, wb2_ref,
                wA_ref, wB_ref, wC_ref, bl_ref, out_ref):
    nrm = 1.0 / math.sqrt(20.0)
    x1T = (x1a_ref[...] + x1b_ref[...]).T
    x2T = (x2a_ref[...] + x2b_ref[...]).T
    aT = (x1T - x2T) * nrm
    bT = (x1T + x2T) * nrm
    xbiT = _bilinear_bf16_t(aT, bT, wb2_ref)
    xT = (jnp.dot(wA_ref[...], aT, preferred_element_type=jnp.float32)
          + jnp.dot(wB_ref[...], bT, preferred_element_type=jnp.float32)
          + jnp.dot(wC_ref[...], xbiT, preferred_element_type=jnp.float32)
          + bl_ref[...])
    out_ref[...] = _colnorm(_silu(xT)).T


def _full(shape):
    return pl.BlockSpec(shape, lambda i: tuple(0 for _ in shape))


def _make_gather(e_tot, chunk):
    mesh = plsc.VectorSubcoreMesh(core_axis_name="c", subcore_axis_name="s")
    nw = 32
    per_w = e_tot // nw
    nch = per_w // chunk

    @functools.partial(
        pl.kernel, mesh=mesh,
        out_type=(jax.ShapeDtypeStruct((e_tot, _D), jnp.float32),
                  jax.ShapeDtypeStruct((e_tot, _D), jnp.float32)),
        scratch_types=[pltpu.VMEM((chunk,), jnp.int32),
                       pltpu.VMEM((chunk, _D), jnp.float32),
                       pltpu.VMEM((chunk,), jnp.int32),
                       pltpu.VMEM((chunk, _D), jnp.float32),
                       pltpu.SemaphoreType.DMA,
                       pltpu.SemaphoreType.DMA],
    )
    def gather(xn_hbm, src_hbm, dst_hbm, outs_hbm, outd_hbm,
               idx1_v, rows1_v, idx2_v, rows2_v, sem1, sem2):
        wid = lax.axis_index("s") * 2 + lax.axis_index("c")
        base = wid * per_w

        def body(k, carry):
            off = base + k * chunk
            pltpu.sync_copy(src_hbm.at[pl.ds(off, chunk)], idx1_v)
            cp1 = pltpu.async_copy(xn_hbm.at[idx1_v], rows1_v, sem1)
            pltpu.sync_copy(dst_hbm.at[pl.ds(off, chunk)], idx2_v)
            cp2 = pltpu.async_copy(xn_hbm.at[idx2_v], rows2_v, sem2)
            cp1.wait()
            pltpu.sync_copy(rows1_v, outs_hbm.at[pl.ds(off, chunk)])
            cp2.wait()
            pltpu.sync_copy(rows2_v, outd_hbm.at[pl.ds(off, chunk)])
            return carry

        lax.fori_loop(0, nch, body, 0)

    return gather


def _make_scatter(e_tot, chunk):
    mesh = plsc.VectorSubcoreMesh(core_axis_name="c", subcore_axis_name="s")
    per_t = e_tot // 16
    nch = per_t // chunk
    nzc = _N // _SCH

    @functools.partial(
        pl.kernel, mesh=mesh,
        out_type=(jax.ShapeDtypeStruct((_N, _D), jnp.float32),
                  jax.ShapeDtypeStruct((_N, _D), jnp.float32)),
        scratch_types=[pltpu.VMEM((chunk,), jnp.int32),
                       pltpu.VMEM((chunk, _D), jnp.float32),
                       pltpu.VMEM_SHARED((_N, _D), jnp.float32)],
    )
    def scatter(y_hbm, dst_hbm, src_hbm, zb_hbm, out1_hbm, out2_hbm,
                idx_v, y_v, acc_sh):
        cid = lax.axis_index("c")
        sid = lax.axis_index("s")

        for t in range((nzc + 15) // 16):
            kk = sid + t * 16

            @pl.when(kk < nzc)
            def _():
                pltpu.sync_copy(zb_hbm, acc_sh.at[pl.ds(kk * _SCH, _SCH)])

        plsc.subcore_barrier()

        def do_chunk(idx_hbm, k):
            off = sid * per_t + k * chunk
            pltpu.sync_copy(idx_hbm.at[pl.ds(off, chunk)], idx_v)
            pltpu.sync_copy(y_hbm.at[pl.ds(off, chunk)], y_v)
            pltpu.sync_copy(y_v, acc_sh.at[idx_v], add=True)

        @pl.when(cid == 0)
        def _():
            lax.fori_loop(0, nch, lambda k, c: (do_chunk(dst_hbm, k), c)[1], 0)

        @pl.when(cid == 1)
        def _():
            lax.fori_loop(0, nch, lambda k, c: (do_chunk(src_hbm, k), c)[1], 0)

        plsc.subcore_barrier()

        for t in range((nzc + 15) // 16):
            kk = sid + t * 16

            @pl.when(kk < nzc)
            def _():
                sl = pl.ds(kk * _SCH, _SCH)

                @pl.when(cid == 0)
                def _():
                    pltpu.sync_copy(acc_sh.at[sl], out1_hbm.at[sl])

                @pl.when(cid == 1)
                def _():
                    pltpu.sync_copy(acc_sh.at[sl], out2_hbm.at[sl])

    return scatter


_EA = 126 * 640
_EB = _E - _EA


@functools.lru_cache(maxsize=1)
def _sc_kernels():
    return (_make_gather(_EA, 168), _make_gather(_EB, 248),
            _make_scatter(_EA, 240), _make_scatter(_EB, 248))


def kernel(xn, xn_attr, xe_attr, xe_src, xe_dst, Wb_xn, Wl_xn, bl_xn,
           W_fc1, b_fc1, Wb_n2e, Wl_n2e, bl_n2e, Wb_xe, Wl_xe, bl_xe,
           W_fc2, b_fc2, Wb_e2n, Wl_e2n, bl_e2n):
    f32 = jnp.float32
    wbxnT = Wb_xn.transpose(2, 1, 0).reshape(_DA * _D, _D)
    wxnA = Wl_xn[:, :_D].T
    wxnB = Wl_xn[:, _D:_D + _DA].T
    wxnC = Wl_xn[:, _D + _DA:].T
    blxn = bl_xn.reshape(1, _D)
    wbn2e2 = Wb_n2e.reshape(_D, _D * _D).astype(jnp.bfloat16)
    wnA = Wl_n2e[:, :_D]
    wnB = Wl_n2e[:, _D:2 * _D]
    wnC = Wl_n2e[:, 2 * _D:]
    bln = bl_n2e.reshape(_D, 1)
    wbxe = Wb_xe[:, :, 0]
    wxA = Wl_xe[:, :_D]
    wxb = Wl_xe[:, _D].reshape(_D, 1)
    wxC = Wl_xe[:, _D + 1:]
    blx = bl_xe.reshape(_D, 1)
    wbe2n2 = Wb_e2n.reshape(_D, _D * _D).astype(jnp.bfloat16)
    weA = Wl_e2n[:, :_D]
    weB = Wl_e2n[:, _D:2 * _D]
    weC = Wl_e2n[:, 2 * _D:]
    ble = bl_e2n.reshape(_D, 1)
    wf1 = W_fc1.reshape(_D, 1)
    bf1 = b_fc1.reshape(_D, 1)
    wf2 = W_fc2.reshape(_D, 1)
    bf2 = b_fc2.reshape(_D, 1)
    ea_row = xe_attr.reshape(1, _E)
    src = xe_src.astype(jnp.int32)
    dst = xe_dst.astype(jnp.int32)

    xn_m = pl.pallas_call(
        _node_mix_body,
        grid=(_N // _BN,),
        in_specs=[pl.BlockSpec((_BN, _D), lambda i: (i, 0)),
                  pl.BlockSpec((_BN, _DA), lambda i: (i, 0)),
                  _full((_DA * _D, _D)), _full((_D, _D)), _full((_DA, _D)),
                  _full((_D, _D)), _full((1, _D))],
        out_specs=pl.BlockSpec((_BN, _D), lambda i: (i, 0)),
        out_shape=jax.ShapeDtypeStruct((_N, _D), f32),
    )(xn, xn_attr, wbxnT, wxnA, wxnB, wxnC, blxn)

    _gatherA, _gatherB, _scatterA, _scatterB = _sc_kernels()
    srcA, srcB = src[:_EA], src[_EA:]
    dstA, dstB = dst[:_EA], dst[_EA:]
    sA, dA = _gatherA(xn_m, srcA, dstA)
    sB, dB = _gatherB(xn_m, srcB, dstB)

    def edge(s_rows, d_rows, ea, ne):
        return pl.pallas_call(
            _edge_body,
            grid=(ne // _BE,),
            in_specs=[pl.BlockSpec((_BE, _D), lambda i: (i, 0)),
                      pl.BlockSpec((_BE, _D), lambda i: (i, 0)),
                      pl.BlockSpec((1, _BE), lambda i: (0, i)),
                      _full((_D, 1)), _full((_D, 1)),
                      _full((_D, _D * _D)),
                      _full((_D, _D)), _full((_D, _D)), _full((_D, _D)),
                      _full((_D, 1)),
                      _full((_D, _D)), _full((_D, _D)), _full((_D, 1)),
                      _full((_D, _D)), _full((_D, 1)),
                      _full((_D, 1)), _full((_D, 1))],
            out_specs=pl.BlockSpec((_BE, _D), lambda i: (i, 0)),
            out_shape=jax.ShapeDtypeStruct((ne, _D), f32),
        )(s_rows, d_rows, ea, wf1, bf1, wbn2e2, wnA, wnB, wnC, bln,
          wbxe, wxA, wxb, wxC, blx, wf2, bf2)

    yA = edge(sA, dA, ea_row[:, :_EA], _EA)
    yB = edge(sB, dB, ea_row[:, _EA:], _EB)

    zb = jnp.zeros((_SCH, _D), f32)
    x1a, x2a = _scatterA(yA, dstA, srcA, zb)
    x1b, x2b = _scatterB(yB, dstB, srcB, zb)

    out = pl.pallas_call(
        _final_body,
        grid=(_N // _BN,),
        in_specs=[pl.BlockSpec((_BN, _D), lambda i: (i, 0)),
                  pl.BlockSpec((_BN, _D), lambda i: (i, 0)),
                  pl.BlockSpec((_BN, _D), lambda i: (i, 0)),
                  pl.BlockSpec((_BN, _D), lambda i: (i, 0)),
                  _full((_D, _D * _D)),
                  _full((_D, _D)), _full((_D, _D)), _full((_D, _D)),
                  _full((_D, 1))],
        out_specs=pl.BlockSpec((_BN, _D), lambda i: (i, 0)),
        out_shape=jax.ShapeDtypeStruct((_N, _D), f32),
    )(x1a, x1b, x2a, x2b, wbe2n2, weA, weB, weC, ble)

    return out


# speedup vs baseline: 1.2867x; 1.0199x over previous
import functools
import math

import jax
import jax.numpy as jnp
from jax import lax
from jax.experimental import pallas as pl
from jax.experimental.pallas import tpu as pltpu
from jax.experimental.pallas import tpu_sc as plsc

_N = 10000
_E = 160000
_D = 128
_DA = 16
_EPS = 1e-9

_BN = 200
_BE = 1280
_GCH = 200
_SCH = 200


def _rownorm(x):
    m = jnp.mean(x, axis=1, keepdims=True)
    var = jnp.sum((x - m) ** 2, axis=1, keepdims=True) / (_D - 1)
    return x / (jnp.sqrt(var) + _EPS)


def _silu(x):
    return x * jax.nn.sigmoid(x)


def _node_mix_body(xn_ref, attr_ref, wbT_ref, wA_ref, wB_ref, wC_ref, bl_ref,
                   out_ref):
    x1 = xn_ref[...]
    x2 = attr_ref[...]
    p = jnp.concatenate([x2[:, j:j + 1] * x1 for j in range(_DA)], axis=1)
    xbi = jnp.dot(p, wbT_ref[...], preferred_element_type=jnp.float32)
    x = (jnp.dot(x1, wA_ref[...], preferred_element_type=jnp.float32)
         + jnp.dot(x2, wB_ref[...], preferred_element_type=jnp.float32)
         + jnp.dot(xbi, wC_ref[...], preferred_element_type=jnp.float32)
         + bl_ref[...])
    out_ref[...] = _rownorm(x)


def _colnorm(x):
    m = jnp.mean(x, axis=0, keepdims=True)
    var = jnp.sum((x - m) ** 2, axis=0, keepdims=True) / (_D - 1)
    return x / (jnp.sqrt(var) + _EPS)


def _bilinear_bf16_t(uT, vT, w2_ref):
    n = uT.shape[1]
    xbiT = jnp.zeros((_D, n), dtype=jnp.float32)
    for g in range(16):
        blocks = [uT[i:i + 1, :] * vT for i in range(8 * g, 8 * g + 8)]
        pT = jnp.concatenate(blocks, axis=0).astype(jnp.bfloat16)
        xbiT += jnp.dot(w2_ref[:, 8 * g * _D:(8 * g + 8) * _D], pT,
                        preferred_element_type=jnp.float32)
    return xbiT


def _edge_body(s_ref, d_ref, ea_ref, wf1_ref, bf1_ref, wb2_ref,
               wnA_ref, wnB_ref, wnC_ref, bln_ref,
               wbxe_ref, wxA_ref, wxb_ref, wxC_ref, blx_ref,
               wf2_ref, bf2_ref, out_ref):
    sT = s_ref[...].T
    dT = d_ref[...].T
    ea = ea_ref[...]
    w = _silu(ea * wf1_ref[...] + bf1_ref[...])
    uT = w * (sT - dT)
    vT = w * (sT + dT) * 0.5
    xbiT = _bilinear_bf16_t(uT, vT, wb2_ref)
    xeT = (jnp.dot(wnA_ref[...], uT, preferred_element_type=jnp.float32)
           + jnp.dot(wnB_ref[...], vT, preferred_element_type=jnp.float32)
           + jnp.dot(wnC_ref[...], xbiT, preferred_element_type=jnp.float32)
           + bln_ref[...])
    xbi2T = jnp.dot(wbxe_ref[...], xeT,
                    preferred_element_type=jnp.float32) * ea
    xe2T = (jnp.dot(wxA_ref[...], xeT, preferred_element_type=jnp.float32)
            + wxb_ref[...] * ea
            + jnp.dot(wxC_ref[...], xbi2T, preferred_element_type=jnp.float32)
            + blx_ref[...])
    xe2T = _colnorm(xe2T)
    w2 = _silu(ea * wf2_ref[...] + bf2_ref[...])
    out_ref[...] = (w2 * xe2T).T


def _final_body(x1a_ref, x1b_ref, x2a_ref, x2b_ref, wb2_ref,
                wA_ref, wB_ref, wC_ref, bl_ref, out_ref):
    nrm = 1.0 / math.sqrt(20.0)
    x1T = (x1a_ref[...] + x1b_ref[...]).T
    x2T = (x2a_ref[...] + x2b_ref[...]).T
    aT = (x1T - x2T) * nrm
    bT = (x1T + x2T) * nrm
    xbiT = _bilinear_bf16_t(aT, bT, wb2_ref)
    xT = (jnp.dot(wA_ref[...], aT, preferred_element_type=jnp.float32)
          + jnp.dot(wB_ref[...], bT, preferred_element_type=jnp.float32)
          + jnp.dot(wC_ref[...], xbiT, preferred_element_type=jnp.float32)
          + bl_ref[...])
    out_ref[...] = _colnorm(_silu(xT)).T


def _full(shape):
    return pl.BlockSpec(shape, lambda i: tuple(0 for _ in shape))


def _make_gather(e_tot, chunk):
    mesh = plsc.VectorSubcoreMesh(core_axis_name="c", subcore_axis_name="s")
    nw = 32
    per_w = e_tot // nw
    nch = per_w // chunk

    @functools.partial(
        pl.kernel, mesh=mesh,
        out_type=(jax.ShapeDtypeStruct((e_tot, _D), jnp.float32),
                  jax.ShapeDtypeStruct((e_tot, _D), jnp.float32)),
        scratch_types=[pltpu.VMEM((chunk,), jnp.int32),
                       pltpu.VMEM((chunk, _D), jnp.float32),
                       pltpu.VMEM((chunk,), jnp.int32),
                       pltpu.VMEM((chunk, _D), jnp.float32),
                       pltpu.SemaphoreType.DMA,
                       pltpu.SemaphoreType.DMA],
    )
    def gather(xn_hbm, src_hbm, dst_hbm, outs_hbm, outd_hbm,
               idx1_v, rows1_v, idx2_v, rows2_v, sem1, sem2):
        wid = lax.axis_index("s") * 2 + lax.axis_index("c")
        base = wid * per_w

        def body(k, carry):
            off = base + k * chunk
            pltpu.sync_copy(src_hbm.at[pl.ds(off, chunk)], idx1_v)
            cp1 = pltpu.async_copy(xn_hbm.at[idx1_v], rows1_v, sem1)
            pltpu.sync_copy(dst_hbm.at[pl.ds(off, chunk)], idx2_v)
            cp2 = pltpu.async_copy(xn_hbm.at[idx2_v], rows2_v, sem2)
            cp1.wait()
            pltpu.sync_copy(rows1_v, outs_hbm.at[pl.ds(off, chunk)])
            cp2.wait()
            pltpu.sync_copy(rows2_v, outd_hbm.at[pl.ds(off, chunk)])
            return carry

        lax.fori_loop(0, nch, body, 0)

    return gather


def _make_scatter(e_p, e_q, chunk):
    mesh = plsc.VectorSubcoreMesh(core_axis_name="c", subcore_axis_name="s")
    per_p = e_p // 16
    nch_p = per_p // chunk
    per_q = e_q // 16
    nch_q = per_q // chunk
    nzc = _N // _SCH

    @functools.partial(
        pl.kernel, mesh=mesh,
        out_type=(jax.ShapeDtypeStruct((_N, _D), jnp.float32),
                  jax.ShapeDtypeStruct((_N, _D), jnp.float32)),
        scratch_types=[pltpu.VMEM((chunk,), jnp.int32),
                       pltpu.VMEM((chunk, _D), jnp.float32),
                       pltpu.VMEM_SHARED((_N, _D), jnp.float32)],
    )
    def scatter(yp_hbm, yq_hbm, dstp_hbm, srcp_hbm, dstq_hbm, srcq_hbm,
                zb_hbm, out1_hbm, out2_hbm, idx_v, y_v, acc_sh):
        cid = lax.axis_index("c")
        sid = lax.axis_index("s")

        for t in range((nzc + 15) // 16):
            kk = sid + t * 16

            @pl.when(kk < nzc)
            def _():
                pltpu.sync_copy(zb_hbm, acc_sh.at[pl.ds(kk * _SCH, _SCH)])

        plsc.subcore_barrier()

        def do_chunk(y_hbm, idx_hbm, per_t, k):
            off = sid * per_t + k * chunk
            pltpu.sync_copy(idx_hbm.at[pl.ds(off, chunk)], idx_v)
            pltpu.sync_copy(y_hbm.at[pl.ds(off, chunk)], y_v)
            pltpu.sync_copy(y_v, acc_sh.at[idx_v], add=True)

        @pl.when(cid == 0)
        def _():
            lax.fori_loop(
                0, nch_p, lambda k, c: (do_chunk(yp_hbm, dstp_hbm, per_p, k), c)[1], 0)
            lax.fori_loop(
                0, nch_q, lambda k, c: (do_chunk(yq_hbm, dstq_hbm, per_q, k), c)[1], 0)

        @pl.when(cid == 1)
        def _():
            lax.fori_loop(
                0, nch_p, lambda k, c: (do_chunk(yp_hbm, srcp_hbm, per_p, k), c)[1], 0)
            lax.fori_loop(
                0, nch_q, lambda k, c: (do_chunk(yq_hbm, srcq_hbm, per_q, k), c)[1], 0)

        plsc.subcore_barrier()

        for t in range((nzc + 15) // 16):
            kk = sid + t * 16

            @pl.when(kk < nzc)
            def _():
                sl = pl.ds(kk * _SCH, _SCH)

                @pl.when(cid == 0)
                def _():
                    pltpu.sync_copy(acc_sh.at[sl], out1_hbm.at[sl])

                @pl.when(cid == 1)
                def _():
                    pltpu.sync_copy(acc_sh.at[sl], out2_hbm.at[sl])

    return scatter


_PARTS = (38400, 38400, 38400, 44800)
_POFF = (0, 38400, 76800, 115200)


@functools.lru_cache(maxsize=1)
def _sc_kernels():
    g1 = _make_gather(38400, 200)
    g2 = _make_gather(44800, 200)
    return ((g1, g1, g1, g2),
            (_make_scatter(38400, 38400, 200),
             _make_scatter(38400, 44800, 200)))


def kernel(xn, xn_attr, xe_attr, xe_src, xe_dst, Wb_xn, Wl_xn, bl_xn,
           W_fc1, b_fc1, Wb_n2e, Wl_n2e, bl_n2e, Wb_xe, Wl_xe, bl_xe,
           W_fc2, b_fc2, Wb_e2n, Wl_e2n, bl_e2n):
    f32 = jnp.float32
    wbxnT = Wb_xn.transpose(2, 1, 0).reshape(_DA * _D, _D)
    wxnA = Wl_xn[:, :_D].T
    wxnB = Wl_xn[:, _D:_D + _DA].T
    wxnC = Wl_xn[:, _D + _DA:].T
    blxn = bl_xn.reshape(1, _D)
    wbn2e2 = Wb_n2e.reshape(_D, _D * _D).astype(jnp.bfloat16)
    wnA = Wl_n2e[:, :_D]
    wnB = Wl_n2e[:, _D:2 * _D]
    wnC = Wl_n2e[:, 2 * _D:]
    bln = bl_n2e.reshape(_D, 1)
    wbxe = Wb_xe[:, :, 0]
    wxA = Wl_xe[:, :_D]
    wxb = Wl_xe[:, _D].reshape(_D, 1)
    wxC = Wl_xe[:, _D + 1:]
    blx = bl_xe.reshape(_D, 1)
    wbe2n2 = Wb_e2n.reshape(_D, _D * _D).astype(jnp.bfloat16)
    weA = Wl_e2n[:, :_D]
    weB = Wl_e2n[:, _D:2 * _D]
    weC = Wl_e2n[:, 2 * _D:]
    ble = bl_e2n.reshape(_D, 1)
    wf1 = W_fc1.reshape(_D, 1)
    bf1 = b_fc1.reshape(_D, 1)
    wf2 = W_fc2.reshape(_D, 1)
    bf2 = b_fc2.reshape(_D, 1)
    ea_row = xe_attr.reshape(1, _E)
    src = xe_src.astype(jnp.int32)
    dst = xe_dst.astype(jnp.int32)

    xn_m = pl.pallas_call(
        _node_mix_body,
        grid=(_N // _BN,),
        in_specs=[pl.BlockSpec((_BN, _D), lambda i: (i, 0)),
                  pl.BlockSpec((_BN, _DA), lambda i: (i, 0)),
                  _full((_DA * _D, _D)), _full((_D, _D)), _full((_DA, _D)),
                  _full((_D, _D)), _full((1, _D))],
        out_specs=pl.BlockSpec((_BN, _D), lambda i: (i, 0)),
        out_shape=jax.ShapeDtypeStruct((_N, _D), f32),
    )(xn, xn_attr, wbxnT, wxnA, wxnB, wxnC, blxn)

    _gathers, _scatters = _sc_kernels()
    srcs = [lax.dynamic_slice_in_dim(src, o, p) for o, p in zip(_POFF, _PARTS)]
    dsts = [lax.dynamic_slice_in_dim(dst, o, p) for o, p in zip(_POFF, _PARTS)]
    rows = [g(xn_m, s_, d_) for g, s_, d_ in zip(_gathers, srcs, dsts)]

    def edge(s_rows, d_rows, ea, ne):
        return pl.pallas_call(
            _edge_body,
            grid=(ne // _BE,),
            in_specs=[pl.BlockSpec((_BE, _D), lambda i: (i, 0)),
                      pl.BlockSpec((_BE, _D), lambda i: (i, 0)),
                      pl.BlockSpec((1, _BE), lambda i: (0, i)),
                      _full((_D, 1)), _full((_D, 1)),
                      _full((_D, _D * _D)),
                      _full((_D, _D)), _full((_D, _D)), _full((_D, _D)),
                      _full((_D, 1)),
                      _full((_D, _D)), _full((_D, _D)), _full((_D, 1)),
                      _full((_D, _D)), _full((_D, 1)),
                      _full((_D, 1)), _full((_D, 1))],
            out_specs=pl.BlockSpec((_BE, _D), lambda i: (i, 0)),
            out_shape=jax.ShapeDtypeStruct((ne, _D), f32),
        )(s_rows, d_rows, ea, wf1, bf1, wbn2e2, wnA, wnB, wnC, bln,
          wbxe, wxA, wxb, wxC, blx, wf2, bf2)

    ys = [edge(s_, d_, lax.dynamic_slice_in_dim(ea_row, o, p, axis=1), p)
          for (s_, d_), o, p in zip(rows, _POFF, _PARTS)]

    zb = jnp.zeros((_SCH, _D), f32)
    x1a, x2a = _scatters[0](ys[0], ys[1], dsts[0], srcs[0], dsts[1], srcs[1],
                            zb)
    x1b, x2b = _scatters[1](ys[2], ys[3], dsts[2], srcs[2], dsts[3], srcs[3],
                            zb)

    out = pl.pallas_call(
        _final_body,
        grid=(_N // _BN,),
        in_specs=[pl.BlockSpec((_BN, _D), lambda i: (i, 0)),
                  pl.BlockSpec((_BN, _D), lambda i: (i, 0)),
                  pl.BlockSpec((_BN, _D), lambda i: (i, 0)),
                  pl.BlockSpec((_BN, _D), lambda i: (i, 0)),
                  _full((_D, _D * _D)),
                  _full((_D, _D)), _full((_D, _D)), _full((_D, _D)),
                  _full((_D, 1))],
        out_specs=pl.BlockSpec((_BN, _D), lambda i: (i, 0)),
        out_shape=jax.ShapeDtypeStruct((_N, _D), f32),
    )(x1a, x1b, x2a, x2b, wbe2n2, weA, weB, weC, ble)

    return out
